# Initial kernel scaffold; baseline (speedup 1.0000x reference)
import jax
import jax.numpy as jnp
from jax.experimental import pallas as pl


def kernel(x, edge_index, W, b):
    raise NotImplementedError("write your pallas kernel here")


# speedup vs baseline: 9.5274x; 9.5274x over previous
import functools

import jax
import jax.numpy as jnp
from jax import lax
from jax.experimental import pallas as pl
from jax.experimental.pallas import tpu as pltpu
from jax.experimental.pallas import tpu_sc as plsc

NC = 2
NS = 16
NT = NC * NS
CHUNK = 128


def _sc_mesh():
    return plsc.VectorSubcoreMesh(
        core_axis_name="c", subcore_axis_name="s", num_cores=NC, num_subcores=NS
    )


def _histogram_call(npad, nch):
    rows_per_tile = npad // NS

    def body(srcdst_hbm, zeros_hbm, deg_hbm, deg_out_sh, deg_in_sh, ones_v, idx_v):
        c = lax.axis_index("c")
        s = lax.axis_index("s")
        wid = s * NC + c
        sl = pl.ds(s * rows_per_tile, rows_per_tile)
        pltpu.sync_copy(zeros_hbm.at[sl], deg_out_sh.at[sl])
        pltpu.sync_copy(zeros_hbm.at[sl], deg_in_sh.at[sl])
        for i in range(CHUNK // 16):
            ones_v[pl.ds(i * 16, 16)] = jnp.ones((16,), jnp.float32)
        plsc.subcore_barrier()
        pltpu.sync_copy(srcdst_hbm.at[0, wid], idx_v)

        def hsrc(j, carry):
            pltpu.sync_copy(ones_v, deg_out_sh.at[idx_v.at[j]], add=True)
            return carry

        lax.fori_loop(0, nch, hsrc, 0)
        pltpu.sync_copy(srcdst_hbm.at[1, wid], idx_v)

        def hdst(j, carry):
            pltpu.sync_copy(ones_v, deg_in_sh.at[idx_v.at[j]], add=True)
            return carry

        lax.fori_loop(0, nch, hdst, 0)
        plsc.subcore_barrier()
        pltpu.sync_copy(deg_out_sh.at[sl], deg_hbm.at[c, sl])
        pltpu.sync_copy(deg_in_sh.at[sl], deg_hbm.at[NC + c, sl])

    return pl.kernel(
        body,
        out_type=jax.ShapeDtypeStruct((2 * NC, npad), jnp.float32),
        mesh=_sc_mesh(),
        scratch_types=[
            pltpu.VMEM_SHARED((npad,), jnp.float32),
            pltpu.VMEM_SHARED((npad,), jnp.float32),
            pltpu.VMEM((CHUNK,), jnp.float32),
            pltpu.VMEM((nch, CHUNK), jnp.int32),
        ],
    )


def _gather_scatter_call(npad, nch, d):
    rows_per_tile = npad // NS

    def body(srcdst_hbm, t_hbm, zrows_hbm, part_hbm, agg_sh, src_idx, dst_idx,
             rows_v, gsem):
        c = lax.axis_index("c")
        s = lax.axis_index("s")
        wid = s * NC + c
        sl = pl.ds(s * rows_per_tile, rows_per_tile)
        pltpu.sync_copy(zrows_hbm.at[sl], agg_sh.at[sl])
        pltpu.sync_copy(srcdst_hbm.at[0, wid], src_idx)
        pltpu.sync_copy(srcdst_hbm.at[1, wid], dst_idx)
        plsc.subcore_barrier()

        def step(j, carry):
            pltpu.async_copy(t_hbm.at[src_idx.at[j]], rows_v, gsem).wait()
            pltpu.sync_copy(rows_v, agg_sh.at[dst_idx.at[j]], add=True)
            return carry

        lax.fori_loop(0, nch, step, 0)
        plsc.subcore_barrier()
        pltpu.sync_copy(agg_sh.at[sl], part_hbm.at[c, sl])

    return pl.kernel(
        body,
        out_type=jax.ShapeDtypeStruct((NC, npad, d), jnp.float32),
        mesh=_sc_mesh(),
        scratch_types=[
            pltpu.VMEM_SHARED((npad, d), jnp.float32),
            pltpu.VMEM((nch, CHUNK), jnp.int32),
            pltpu.VMEM((nch, CHUNK), jnp.int32),
            pltpu.VMEM((CHUNK, d), jnp.float32),
            pltpu.SemaphoreType.DMA,
        ],
    )


def _norm_matmul(x_pad, w, deg, npad, d, blk=1024):
    def body(x_ref, w_ref, deg_ref, t_ref):
        dv = deg_ref[0, :] + deg_ref[1, :]
        norm = jnp.where(dv > 0, lax.rsqrt(jnp.maximum(dv, 1e-12)), 0.0)
        t_ref[...] = jnp.dot(
            x_ref[...] * norm[:, None], w_ref[...],
            preferred_element_type=jnp.float32,
        )

    return pl.pallas_call(
        body,
        grid=(npad // blk,),
        in_specs=[
            pl.BlockSpec((blk, d), lambda i: (i, 0)),
            pl.BlockSpec((d, d), lambda i: (0, 0)),
            pl.BlockSpec((2 * NC, blk), lambda i: (0, i)),
        ],
        out_specs=pl.BlockSpec((blk, d), lambda i: (i, 0)),
        out_shape=jax.ShapeDtypeStruct((npad, d), jnp.float32),
    )(x_pad, w, deg)


def _finalize(parts, deg, b2d, npad, d, blk=1024):
    def body(part_ref, deg_ref, b_ref, out_ref):
        ssum = part_ref[0] + part_ref[1]
        dv = deg_ref[NC, :] + deg_ref[NC + 1, :]
        norm = jnp.where(dv > 0, lax.rsqrt(jnp.maximum(dv, 1e-12)), 0.0)
        out_ref[...] = ssum * norm[:, None] + b_ref[0, :][None, :]

    return pl.pallas_call(
        body,
        grid=(npad // blk,),
        in_specs=[
            pl.BlockSpec((NC, blk, d), lambda i: (0, i, 0)),
            pl.BlockSpec((2 * NC, blk), lambda i: (0, i)),
            pl.BlockSpec((1, d), lambda i: (0, 0)),
        ],
        out_specs=pl.BlockSpec((blk, d), lambda i: (i, 0)),
        out_shape=jax.ShapeDtypeStruct((npad, d), jnp.float32),
    )(parts, deg, b2d)


def kernel(x, edge_index, W, b):
    n, d = x.shape
    e = edge_index.shape[1]
    npad = ((n + NS * 8) // (NS * 8) + 1) * (NS * 8)
    epad = ((e + NT * CHUNK - 1) // (NT * CHUNK)) * (NT * CHUNK)
    nch = epad // (NT * CHUNK)

    pad_ids = n + (jnp.arange(epad - e, dtype=jnp.int32) % (npad - n))
    src = jnp.concatenate([edge_index[0], pad_ids])
    dst = jnp.concatenate([edge_index[1], pad_ids])
    srcdst = jnp.stack([src, dst]).reshape(2, NT, nch, CHUNK)

    x_pad = jnp.concatenate([x, jnp.zeros((npad - n, d), jnp.float32)])
    zeros_n = jnp.zeros((npad,), jnp.float32)
    zrows = jnp.zeros((npad, d), jnp.float32)

    deg = _histogram_call(npad, nch)(srcdst, zeros_n)
    t = _norm_matmul(x_pad, W, deg, npad, d)
    parts = _gather_scatter_call(npad, nch, d)(srcdst, t, zrows)
    out_pad = _finalize(parts, deg, b.reshape(1, d), npad, d)
    return out_pad[:n]


# speedup vs baseline: 12.2975x; 1.2908x over previous
import jax
import jax.numpy as jnp
from jax import lax
from jax.experimental import pallas as pl
from jax.experimental.pallas import tpu as pltpu
from jax.experimental.pallas import tpu_sc as plsc

NC = 2
NS = 16
NT = NC * NS
CHUNK = 80


def _sc_mesh():
    return plsc.VectorSubcoreMesh(
        core_axis_name="c", subcore_axis_name="s", num_cores=NC, num_subcores=NS
    )


def _histogram_call(npad, nch):
    rows_per_tile = npad // NS

    def body(srcdst_hbm, zeros_hbm, deg_hbm, deg_out_sh, deg_in_sh, ones_v, idx_v):
        c = lax.axis_index("c")
        s = lax.axis_index("s")
        wid = s * NC + c
        sl = pl.ds(s * rows_per_tile, rows_per_tile)
        pltpu.sync_copy(zeros_hbm.at[sl], deg_out_sh.at[sl])
        pltpu.sync_copy(zeros_hbm.at[sl], deg_in_sh.at[sl])
        for i in range(CHUNK // 16):
            ones_v[pl.ds(i * 16, 16)] = jnp.ones((16,), jnp.float32)
        plsc.subcore_barrier()
        pltpu.sync_copy(srcdst_hbm.at[0, wid], idx_v)

        def hsrc(j, carry):
            pltpu.sync_copy(ones_v, deg_out_sh.at[idx_v.at[j]], add=True)
            return carry

        lax.fori_loop(0, nch, hsrc, 0)
        pltpu.sync_copy(srcdst_hbm.at[1, wid], idx_v)

        def hdst(j, carry):
            pltpu.sync_copy(ones_v, deg_in_sh.at[idx_v.at[j]], add=True)
            return carry

        lax.fori_loop(0, nch, hdst, 0)
        plsc.subcore_barrier()
        base = s * rows_per_tile
        pltpu.sync_copy(deg_out_sh.at[sl],
                        deg_hbm.at[pl.ds(c * npad + base, rows_per_tile)])
        pltpu.sync_copy(deg_in_sh.at[sl],
                        deg_hbm.at[pl.ds((NC + c) * npad + base, rows_per_tile)])

    return pl.kernel(
        body,
        out_type=jax.ShapeDtypeStruct((2 * NC * npad,), jnp.float32),
        mesh=_sc_mesh(),
        scratch_types=[
            pltpu.VMEM_SHARED((npad,), jnp.float32),
            pltpu.VMEM_SHARED((npad,), jnp.float32),
            pltpu.VMEM((CHUNK,), jnp.float32),
            pltpu.VMEM((nch, CHUNK), jnp.int32),
        ],
    )


def _gather_scatter_call(npad, nch, d):
    rows_per_tile = npad // NS

    def body(srcdst_hbm, t_hbm, zrows_hbm, part_hbm, agg_sh, src_idx,
             dstb0, dstb1, rows0, rows1, gsem0, gsem1, dsem0, dsem1):
        c = lax.axis_index("c")
        s = lax.axis_index("s")
        wid = s * NC + c
        sl = pl.ds(s * rows_per_tile, rows_per_tile)
        pltpu.sync_copy(zrows_hbm.at[sl], agg_sh.at[sl])
        pltpu.sync_copy(srcdst_hbm.at[0, wid], src_idx)
        plsc.subcore_barrier()

        slots = ((rows0, gsem0, dstb0, dsem0), (rows1, gsem1, dstb1, dsem1))
        for j0, (buf, sem, dstb, dsem) in enumerate(slots):
            pltpu.async_copy(srcdst_hbm.at[1, wid, pl.ds(j0, 1)], dstb, dsem)
            pltpu.async_copy(t_hbm.at[src_idx.at[j0]], buf, sem)

        def step(g, carry):
            for slot, (buf, sem, dstb, dsem) in enumerate(slots):
                j = g * 2 + slot
                pltpu.make_async_copy(srcdst_hbm.at[1, wid, pl.ds(j, 1)], dstb, dsem).wait()
                pltpu.make_async_copy(t_hbm.at[src_idx.at[j]], buf, sem).wait()
                pltpu.sync_copy(buf, agg_sh.at[dstb.at[0]], add=True)

                @pl.when(j + 2 < nch)
                def _():
                    pltpu.async_copy(srcdst_hbm.at[1, wid, pl.ds(j + 2, 1)], dstb, dsem)
                    pltpu.async_copy(t_hbm.at[src_idx.at[j + 2]], buf, sem)

            return carry

        lax.fori_loop(0, nch // 2, step, 0)
        if nch % 2:
            j = nch - 1
            buf, sem, dstb, dsem = slots[0]
            pltpu.make_async_copy(srcdst_hbm.at[1, wid, pl.ds(j, 1)], dstb, dsem).wait()
            pltpu.make_async_copy(t_hbm.at[src_idx.at[j]], buf, sem).wait()
            pltpu.sync_copy(buf, agg_sh.at[dstb.at[0]], add=True)
        plsc.subcore_barrier()
        pltpu.sync_copy(agg_sh.at[sl], part_hbm.at[c, sl])

    return pl.kernel(
        body,
        out_type=jax.ShapeDtypeStruct((NC, npad, d), jnp.float32),
        mesh=_sc_mesh(),
        scratch_types=[
            pltpu.VMEM_SHARED((npad, d), jnp.float32),
            pltpu.VMEM((nch, CHUNK), jnp.int32),
            pltpu.VMEM((1, CHUNK), jnp.int32),
            pltpu.VMEM((1, CHUNK), jnp.int32),
            pltpu.VMEM((CHUNK, d), jnp.float32),
            pltpu.VMEM((CHUNK, d), jnp.float32),
            pltpu.SemaphoreType.DMA,
            pltpu.SemaphoreType.DMA,
            pltpu.SemaphoreType.DMA,
            pltpu.SemaphoreType.DMA,
        ],
    )


def _norm_matmul(x_in, w, deg, d, blk):
    n = x_in.shape[0]

    def body(x_ref, w_ref, deg_ref, t_ref):
        dv = deg_ref[:, 0] + deg_ref[:, 1]
        norm = jnp.where(dv > 0, lax.rsqrt(jnp.maximum(dv, 1e-12)), 0.0)
        t_ref[...] = jnp.dot(
            x_ref[...] * norm[:, None], w_ref[...],
            preferred_element_type=jnp.float32,
        )

    return pl.pallas_call(
        body,
        grid=(n // blk,),
        in_specs=[
            pl.BlockSpec((blk, d), lambda i: (i, 0)),
            pl.BlockSpec((d, d), lambda i: (0, 0)),
            pl.BlockSpec((blk, 2 * NC), lambda i: (i, 0)),
        ],
        out_specs=pl.BlockSpec((blk, d), lambda i: (i, 0)),
        out_shape=jax.ShapeDtypeStruct((n, d), jnp.float32),
    )(x_in, w, deg)


def _finalize(parts, deg, b2d, n, d, blk):
    def body(part_ref, deg_ref, b_ref, out_ref):
        ssum = part_ref[0] + part_ref[1]
        dv = deg_ref[:, NC] + deg_ref[:, NC + 1]
        norm = jnp.where(dv > 0, lax.rsqrt(jnp.maximum(dv, 1e-12)), 0.0)
        out_ref[...] = ssum * norm[:, None] + b_ref[0, :][None, :]

    return pl.pallas_call(
        body,
        grid=(n // blk,),
        in_specs=[
            pl.BlockSpec((NC, blk, d), lambda i: (0, i, 0)),
            pl.BlockSpec((blk, 2 * NC), lambda i: (i, 0)),
            pl.BlockSpec((1, d), lambda i: (0, 0)),
        ],
        out_specs=pl.BlockSpec((blk, d), lambda i: (i, 0)),
        out_shape=jax.ShapeDtypeStruct((n, d), jnp.float32),
    )(parts, deg, b2d)


def kernel(x, edge_index, W, b):
    n, d = x.shape
    e = edge_index.shape[1]
    npad = ((n + NS * 16 - 1) // (NS * 16)) * (NS * 16)
    epad = ((e + NT * CHUNK - 1) // (NT * CHUNK)) * (NT * CHUNK)
    nch = epad // (NT * CHUNK)

    if epad > e:
        npad = npad + NS * 16 if npad == n else npad
        spread = max(npad - n, 1)
        pad_ids = n + (jnp.arange(epad - e, dtype=jnp.int32) % spread)
        srcdst = jnp.concatenate(
            [edge_index, jnp.broadcast_to(pad_ids, (2, epad - e))], axis=1
        ).reshape(2, NT, nch, CHUNK)
        x_in = jnp.concatenate([x, jnp.zeros((npad - n, d), jnp.float32)])
    else:
        srcdst = edge_index.reshape(2, NT, nch, CHUNK)
        x_in = x

    nt = x_in.shape[0]
    zeros_n = jnp.zeros((npad,), jnp.float32)
    zrows = jnp.zeros((npad, d), jnp.float32)

    deg = _histogram_call(npad, nch)(srcdst, zeros_n)
    deg_t = deg.reshape(2 * NC, npad).T
    t = _norm_matmul(x_in, W, deg_t, d, blk=nt // 10 if nt % 10 == 0 else nt)
    parts = _gather_scatter_call(npad, nch, d)(srcdst, t, zrows)
    out = _finalize(parts, deg_t, b.reshape(1, d), n, d,
                    blk=n // 10 if n % 10 == 0 else n)
    return out


# speedup vs baseline: 13.5131x; 1.0988x over previous
import jax
import jax.numpy as jnp
from jax import lax
from jax.experimental import pallas as pl
from jax.experimental.pallas import tpu as pltpu
from jax.experimental.pallas import tpu_sc as plsc

NC = 2
NS = 16
NT = NC * NS
CHUNK = 80


def _sc_mesh():
    return plsc.VectorSubcoreMesh(
        core_axis_name="c", subcore_axis_name="s", num_cores=NC, num_subcores=NS
    )


def _histogram_call(npad, nch):
    rows_per_tile = npad // NS

    WIN = 8

    def body(srcdst_hbm, zeros_hbm, deg_hbm, deg_out_sh, deg_in_sh, ones_v,
             idx_src, idx_dst, hsem):
        c = lax.axis_index("c")
        s = lax.axis_index("s")
        wid = s * NC + c
        sl = pl.ds(s * rows_per_tile, rows_per_tile)
        pltpu.sync_copy(zeros_hbm.at[sl], deg_out_sh.at[sl])
        pltpu.sync_copy(zeros_hbm.at[sl], deg_in_sh.at[sl])
        for i in range(CHUNK // 16):
            ones_v[pl.ds(i * 16, 16)] = jnp.ones((16,), jnp.float32)
        plsc.subcore_barrier()
        pltpu.sync_copy(srcdst_hbm.at[0, wid], idx_src)
        pltpu.sync_copy(srcdst_hbm.at[1, wid], idx_dst)

        def hsrc(j, carry):
            pltpu.sync_copy(ones_v, deg_out_sh.at[idx_src.at[j]], add=True)
            return carry

        lax.fori_loop(0, nch, hsrc, 0)

        def hdst(j, carry):
            pltpu.sync_copy(ones_v, deg_in_sh.at[idx_dst.at[j]], add=True)
            return carry

        lax.fori_loop(0, nch, hdst, 0)
        plsc.subcore_barrier()
        base = s * rows_per_tile
        pltpu.sync_copy(deg_out_sh.at[sl],
                        deg_hbm.at[pl.ds(c * npad + base, rows_per_tile)])
        pltpu.sync_copy(deg_in_sh.at[sl],
                        deg_hbm.at[pl.ds((NC + c) * npad + base, rows_per_tile)])

    return pl.kernel(
        body,
        out_type=jax.ShapeDtypeStruct((2 * NC * npad,), jnp.float32),
        mesh=_sc_mesh(),
        scratch_types=[
            pltpu.VMEM_SHARED((npad,), jnp.float32),
            pltpu.VMEM_SHARED((npad,), jnp.float32),
            pltpu.VMEM((CHUNK,), jnp.float32),
            pltpu.VMEM((nch, CHUNK), jnp.int32),
            pltpu.VMEM((nch, CHUNK), jnp.int32),
            pltpu.SemaphoreType.DMA,
        ],
    )


def _gather_scatter_call(npad, nch, d):
    rows_per_tile = npad // NS

    NB = 3

    def body(srcdst_hbm, t_hbm, zrows_hbm, part_hbm, agg_sh, src_idx,
             dstb0, dstb1, dstb2, rows0, rows1, rows2,
             gsem0, gsem1, gsem2, dsem0, dsem1, dsem2, ssem0, ssem1, ssem2):
        c = lax.axis_index("c")
        s = lax.axis_index("s")
        wid = s * NC + c
        sl = pl.ds(s * rows_per_tile, rows_per_tile)
        pltpu.sync_copy(zrows_hbm.at[sl], agg_sh.at[sl])
        pltpu.sync_copy(srcdst_hbm.at[0, wid], src_idx)
        plsc.subcore_barrier()

        rows = (rows0, rows1, rows2)
        gsems = (gsem0, gsem1, gsem2)
        dstbs = (dstb0, dstb1, dstb2)
        dsems = (dsem0, dsem1, dsem2)
        ssems = (ssem0, ssem1, ssem2)

        def issue(j, slot):
            pltpu.async_copy(srcdst_hbm.at[1, wid, pl.ds(j, 1)],
                             dstbs[slot], dsems[slot])
            pltpu.async_copy(t_hbm.at[src_idx.at[j]], rows[slot], gsems[slot])

        def wait_in(j, slot):
            pltpu.make_async_copy(srcdst_hbm.at[1, wid, pl.ds(j, 1)],
                                  dstbs[slot], dsems[slot]).wait()
            pltpu.make_async_copy(t_hbm.at[src_idx.at[j]],
                                  rows[slot], gsems[slot]).wait()

        def wait_scat(slot):
            pltpu.make_async_copy(rows[slot], agg_sh.at[dstbs[slot].at[0]],
                                  ssems[slot]).wait()

        issue(0, 0)
        issue(1, 1)

        def step(g, carry):
            for slot in range(NB):
                j = g * NB + slot
                wait_in(j, slot)
                pltpu.async_copy(rows[slot], agg_sh.at[dstbs[slot].at[0]],
                                 ssems[slot], add=True)
                nslot = (slot + 2) % NB

                @pl.when(j + 2 < nch)
                def _():
                    @pl.when(j >= 1)
                    def _():
                        wait_scat(nslot)

                    issue(j + 2, nslot)

            return carry

        lax.fori_loop(0, nch // NB, step, 0)
        for j in range((nch // NB) * NB, nch):
            slot = j % NB
            wait_in(j, slot)
            pltpu.async_copy(rows[slot], agg_sh.at[dstbs[slot].at[0]],
                             ssems[slot], add=True)
            if j + 2 < nch:
                wait_scat((slot + 2) % NB)
                issue(j + 2, (slot + 2) % NB)
        for j in range(max(nch - NB, 0), nch):
            wait_scat(j % NB)
        plsc.subcore_barrier()
        pltpu.sync_copy(agg_sh.at[sl], part_hbm.at[c, sl])

    return pl.kernel(
        body,
        out_type=jax.ShapeDtypeStruct((NC, npad, d), jnp.float32),
        mesh=_sc_mesh(),
        scratch_types=[
            pltpu.VMEM_SHARED((npad, d), jnp.float32),
            pltpu.VMEM((nch, CHUNK), jnp.int32),
            pltpu.VMEM((1, CHUNK), jnp.int32),
            pltpu.VMEM((1, CHUNK), jnp.int32),
            pltpu.VMEM((1, CHUNK), jnp.int32),
            pltpu.VMEM((CHUNK, d), jnp.float32),
            pltpu.VMEM((CHUNK, d), jnp.float32),
            pltpu.VMEM((CHUNK, d), jnp.float32),
            pltpu.SemaphoreType.DMA,
            pltpu.SemaphoreType.DMA,
            pltpu.SemaphoreType.DMA,
            pltpu.SemaphoreType.DMA,
            pltpu.SemaphoreType.DMA,
            pltpu.SemaphoreType.DMA,
            pltpu.SemaphoreType.DMA,
            pltpu.SemaphoreType.DMA,
            pltpu.SemaphoreType.DMA,
        ],
    )


def _norm_matmul(x_in, w, deg, d, blk):
    n = x_in.shape[0]

    def body(x_ref, w_ref, deg_ref, t_ref):
        dv = deg_ref[:, 0] + deg_ref[:, 1]
        norm = jnp.where(dv > 0, lax.rsqrt(jnp.maximum(dv, 1e-12)), 0.0)
        t_ref[...] = jnp.dot(
            x_ref[...] * norm[:, None], w_ref[...],
            preferred_element_type=jnp.float32,
        )

    return pl.pallas_call(
        body,
        grid=(n // blk,),
        in_specs=[
            pl.BlockSpec((blk, d), lambda i: (i, 0)),
            pl.BlockSpec((d, d), lambda i: (0, 0)),
            pl.BlockSpec((blk, 2 * NC), lambda i: (i, 0)),
        ],
        out_specs=pl.BlockSpec((blk, d), lambda i: (i, 0)),
        out_shape=jax.ShapeDtypeStruct((n, d), jnp.float32),
    )(x_in, w, deg)


def _finalize(parts, deg, b2d, n, d, blk):
    def body(part_ref, deg_ref, b_ref, out_ref):
        ssum = part_ref[0] + part_ref[1]
        dv = deg_ref[:, NC] + deg_ref[:, NC + 1]
        norm = jnp.where(dv > 0, lax.rsqrt(jnp.maximum(dv, 1e-12)), 0.0)
        out_ref[...] = ssum * norm[:, None] + b_ref[0, :][None, :]

    return pl.pallas_call(
        body,
        grid=(n // blk,),
        in_specs=[
            pl.BlockSpec((NC, blk, d), lambda i: (0, i, 0)),
            pl.BlockSpec((blk, 2 * NC), lambda i: (i, 0)),
            pl.BlockSpec((1, d), lambda i: (0, 0)),
        ],
        out_specs=pl.BlockSpec((blk, d), lambda i: (i, 0)),
        out_shape=jax.ShapeDtypeStruct((n, d), jnp.float32),
    )(parts, deg, b2d)


def kernel(x, edge_index, W, b):
    n, d = x.shape
    e = edge_index.shape[1]
    npad = ((n + NS * 16 - 1) // (NS * 16)) * (NS * 16)
    epad = ((e + NT * CHUNK - 1) // (NT * CHUNK)) * (NT * CHUNK)
    nch = epad // (NT * CHUNK)

    if epad > e:
        npad = npad + NS * 16 if npad == n else npad
        spread = max(npad - n, 1)
        pad_ids = n + (jnp.arange(epad - e, dtype=jnp.int32) % spread)
        srcdst = jnp.concatenate(
            [edge_index, jnp.broadcast_to(pad_ids, (2, epad - e))], axis=1
        ).reshape(2, NT, nch, CHUNK)
        x_in = jnp.concatenate([x, jnp.zeros((npad - n, d), jnp.float32)])
    else:
        srcdst = edge_index.reshape(2, NT, nch, CHUNK)
        x_in = x

    nt = x_in.shape[0]
    zeros_n = jnp.zeros((npad,), jnp.float32)
    zrows = jnp.zeros((npad, d), jnp.float32)

    deg = _histogram_call(npad, nch)(srcdst, zeros_n)
    deg_t = deg.reshape(2 * NC, npad).T
    t = _norm_matmul(x_in, W, deg_t, d, blk=nt // 10 if nt % 10 == 0 else nt)
    parts = _gather_scatter_call(npad, nch, d)(srcdst, t, zrows)
    out = _finalize(parts, deg_t, b.reshape(1, d), n, d,
                    blk=n // 10 if n % 10 == 0 else n)
    return out
